# baseline (speedup 1.0000x reference)
import math

import jax
import jax.numpy as jnp
from jax import lax
from jax.experimental import pallas as pl
from jax.experimental.pallas import tpu as pltpu
from jax.experimental.pallas import tpu_sc as plsc

DIM = 128
B = 1024
L = 200
N_TOK = B * L
NC, NS = 2, 16
NW = NC * NS
PER_W = N_TOK // NW
CHUNK = 100
N_CHUNKS = PER_W // CHUNK
NBUF = 4
UNROLL = 8
SCALE = math.sqrt(DIM)
assert N_CHUNKS % NBUF == 0 and NBUF % (L // CHUNK) == 0


def _sc_embed(idx2d, lut, pe2d):
    mesh = plsc.VectorSubcoreMesh(core_axis_name="c", subcore_axis_name="s")

    def body(idx_hbm, lut_hbm, pe_hbm, out_hbm, *scratch):
        idx_v, pe_v = scratch[0], scratch[1]
        gb = scratch[2:2 + NBUF]
        gsem = scratch[2 + NBUF:2 + 2 * NBUF]
        ssem = scratch[2 + 2 * NBUF:2 + 3 * NBUF]
        wid = lax.axis_index("s") * NC + lax.axis_index("c")
        pltpu.sync_copy(idx_hbm.at[pl.ds(wid * N_CHUNKS, N_CHUNKS)], idx_v)
        pltpu.sync_copy(pe_hbm, pe_v)

        def start_gather(j, b):
            pltpu.async_copy(lut_hbm.at[idx_v.at[j]], gb[b], gsem[b])

        def wait_gather(b):
            pltpu.make_async_copy(
                lut_hbm.at[idx_v.at[0]], gb[b], gsem[b]).wait()

        def start_scatter(j, b):
            pltpu.async_copy(
                gb[b], out_hbm.at[wid * N_CHUNKS + j], ssem[b])

        def wait_scatter(b):
            pltpu.make_async_copy(gb[b], out_hbm.at[0], ssem[b]).wait()

        def compute(b):
            ph = (b % 2) * CHUNK
            buf = gb[b]

            @plsc.parallel_loop(0, CHUNK, unroll=UNROLL)
            def _(r):
                for v in range(DIM // 16):
                    sl = pl.ds(v * 16, 16)
                    buf[r, sl] = buf[r, sl] * SCALE + pe_v[ph + r, sl]

        for j in range(NBUF - 1):
            start_gather(j, j)

        def ring_body(jr, carry):
            for b in range(NBUF):
                j = jr * NBUF + b
                nb = (b + NBUF - 1) % NBUF
                @pl.when(j >= 1)
                def _():
                    wait_scatter(nb)

                @pl.when(j + NBUF - 1 < N_CHUNKS)
                def _():
                    start_gather(j + NBUF - 1, nb)

                wait_gather(b)
                compute(b)
                start_scatter(j, b)
            return carry

        lax.fori_loop(0, N_CHUNKS // NBUF, ring_body, 0)
        wait_scatter((N_CHUNKS - 1) % NBUF)

    run = pl.kernel(
        body,
        out_type=jax.ShapeDtypeStruct((N_TOK // CHUNK, CHUNK, DIM),
                                      jnp.float32),
        mesh=mesh,
        scratch_types=(
            [pltpu.VMEM((N_CHUNKS, CHUNK), jnp.int32),
             pltpu.VMEM((L, DIM), jnp.float32)]
            + [pltpu.VMEM((CHUNK, DIM), jnp.float32)] * NBUF
            + [pltpu.SemaphoreType.DMA] * (2 * NBUF)
        ),
    )
    return run(idx2d, lut, pe2d)


@jax.jit
def kernel(tensor, lut, pe):
    idx2d = tensor.reshape(N_TOK // CHUNK, CHUNK)
    pe2d = pe[0, :L, :]
    out = _sc_embed(idx2d, lut, pe2d)
    return out.reshape(B, L, DIM)


# speedup vs baseline: 1.6629x; 1.6629x over previous
import math

import jax
import jax.numpy as jnp
from jax import lax
from jax.experimental import pallas as pl
from jax.experimental.pallas import tpu as pltpu
from jax.experimental.pallas import tpu_sc as plsc

DIM = 128
B = 1024
L = 200
N_TOK = B * L
NC, NS = 2, 16
NW = NC * NS
PER_W = N_TOK // NW
CHUNK = 64
N_CHUNKS = PER_W // CHUNK
NBUF = 10
UNROLL = 8
SCALE = math.sqrt(DIM)
assert N_CHUNKS % NBUF == 0


def _sc_embed(idx_flat, lut, pe2d):
    mesh = plsc.VectorSubcoreMesh(core_axis_name="c", subcore_axis_name="s")

    def body(idx_hbm, lut_hbm, pe_hbm, out_hbm, *scratch):
        idx_v, pe_v = scratch[0], scratch[1]
        gb = scratch[2:2 + NBUF]
        gsem = scratch[2 + NBUF:2 + 2 * NBUF]
        ssem = scratch[2 + 2 * NBUF:2 + 3 * NBUF]
        wid = lax.axis_index("s") * NC + lax.axis_index("c")
        base = wid * PER_W
        pltpu.sync_copy(idx_hbm.at[pl.ds(base, PER_W)], idx_v)
        pltpu.sync_copy(pe_hbm, pe_v)

        def start_gather(j, b):
            pltpu.async_copy(
                lut_hbm.at[idx_v.at[pl.ds(j * CHUNK, CHUNK)]], gb[b], gsem[b])

        def wait_gather(b):
            pltpu.make_async_copy(
                lut_hbm.at[idx_v.at[pl.ds(0, CHUNK)]], gb[b], gsem[b]).wait()

        def start_scatter(j, b):
            pltpu.async_copy(
                gb[b], out_hbm.at[pl.ds(base + j * CHUNK, CHUNK)], ssem[b])

        def wait_scatter(b):
            pltpu.make_async_copy(
                gb[b], out_hbm.at[pl.ds(base, CHUNK)], ssem[b]).wait()

        def compute(j, b):
            ph = lax.rem(j * CHUNK, L)
            buf = gb[b]

            @plsc.parallel_loop(0, CHUNK, unroll=UNROLL)
            def _(r):
                lrow = ph + r
                lrow = jnp.where(lrow >= L, lrow - L, lrow)
                for v in range(DIM // 16):
                    sl = pl.ds(v * 16, 16)
                    buf[r, sl] = buf[r, sl] * SCALE + pe_v[lrow, sl]

        for j in range(NBUF - 1):
            start_gather(j, j)

        def ring_body(jr, carry):
            for b in range(NBUF):
                j = jr * NBUF + b
                nb = (b + NBUF - 1) % NBUF
                @pl.when(j >= 1)
                def _():
                    wait_scatter(nb)

                @pl.when(j + NBUF - 1 < N_CHUNKS)
                def _():
                    start_gather(j + NBUF - 1, nb)

                wait_gather(b)
                compute(j, b)
                start_scatter(j, b)
            return carry

        lax.fori_loop(0, N_CHUNKS // NBUF, ring_body, 0)
        wait_scatter((N_CHUNKS - 1) % NBUF)

    run = pl.kernel(
        body,
        out_type=jax.ShapeDtypeStruct((N_TOK, DIM), jnp.float32),
        mesh=mesh,
        scratch_types=(
            [pltpu.VMEM((PER_W,), jnp.int32),
             pltpu.VMEM((L, DIM), jnp.float32)]
            + [pltpu.VMEM((CHUNK, DIM), jnp.float32)] * NBUF
            + [pltpu.SemaphoreType.DMA] * (2 * NBUF)
        ),
    )
    return run(idx_flat, lut, pe2d)


@jax.jit
def kernel(tensor, lut, pe):
    idx_flat = tensor.reshape(N_TOK)
    pe2d = pe[0, :L, :]
    out = _sc_embed(idx_flat, lut, pe2d)
    return out.reshape(B, L, DIM)


# speedup vs baseline: 2.1222x; 1.2762x over previous
import math

import jax
import jax.numpy as jnp
from jax import lax
from jax.experimental import pallas as pl
from jax.experimental.pallas import tpu as pltpu
from jax.experimental.pallas import tpu_sc as plsc

DIM = 128
B = 1024
L = 200
N_TOK = B * L
NC, NS = 2, 16
NW = NC * NS
PER_W = N_TOK // NW
CHUNK = 128
N_CHUNKS = PER_W // CHUNK
NBUF = 5
UNROLL = 8
SCALE = math.sqrt(DIM)
assert N_CHUNKS % NBUF == 0


def _sc_embed(idx_flat, lut, pe2d):
    mesh = plsc.VectorSubcoreMesh(core_axis_name="c", subcore_axis_name="s")

    def body(idx_hbm, lut_hbm, pe_hbm, out_hbm, *scratch):
        idx_v, pe_v = scratch[0], scratch[1]
        gb = scratch[2:2 + NBUF]
        gsem = scratch[2 + NBUF:2 + 2 * NBUF]
        ssem = scratch[2 + 2 * NBUF:2 + 3 * NBUF]
        wid = lax.axis_index("s") * NC + lax.axis_index("c")
        base = wid * PER_W
        pltpu.sync_copy(idx_hbm.at[pl.ds(base, PER_W)], idx_v)
        pltpu.sync_copy(pe_hbm, pe_v)

        def start_gather(j, b):
            pltpu.async_copy(
                lut_hbm.at[idx_v.at[pl.ds(j * CHUNK, CHUNK)]], gb[b], gsem[b])

        def wait_gather(b):
            pltpu.make_async_copy(
                lut_hbm.at[idx_v.at[pl.ds(0, CHUNK)]], gb[b], gsem[b]).wait()

        def start_scatter(j, b):
            pltpu.async_copy(
                gb[b], out_hbm.at[pl.ds(base + j * CHUNK, CHUNK)], ssem[b])

        def wait_scatter(b):
            pltpu.make_async_copy(
                gb[b], out_hbm.at[pl.ds(base, CHUNK)], ssem[b]).wait()

        def compute(j, b):
            ph = lax.rem(j * CHUNK, L)
            buf = gb[b]

            @plsc.parallel_loop(0, CHUNK, unroll=UNROLL)
            def _(r):
                lrow = ph + r
                lrow = jnp.where(lrow >= L, lrow - L, lrow)
                for v in range(DIM // 16):
                    sl = pl.ds(v * 16, 16)
                    buf[r, sl] = buf[r, sl] * SCALE + pe_v[lrow, sl]

        for j in range(NBUF - 1):
            start_gather(j, j)

        def ring_body(jr, carry):
            for b in range(NBUF):
                j = jr * NBUF + b
                nb = (b + NBUF - 1) % NBUF
                wait_gather(b)
                compute(j, b)

                @pl.when(j >= 1)
                def _():
                    wait_scatter(nb)

                @pl.when(j + NBUF - 1 < N_CHUNKS)
                def _():
                    start_gather(j + NBUF - 1, nb)

                start_scatter(j, b)
            return carry

        lax.fori_loop(0, N_CHUNKS // NBUF, ring_body, 0)
        wait_scatter((N_CHUNKS - 1) % NBUF)

    run = pl.kernel(
        body,
        out_type=jax.ShapeDtypeStruct((N_TOK, DIM), jnp.float32),
        mesh=mesh,
        scratch_types=(
            [pltpu.VMEM((PER_W,), jnp.int32),
             pltpu.VMEM((L, DIM), jnp.float32)]
            + [pltpu.VMEM((CHUNK, DIM), jnp.float32)] * NBUF
            + [pltpu.SemaphoreType.DMA] * (2 * NBUF)
        ),
    )
    return run(idx_flat, lut, pe2d)


@jax.jit
def kernel(tensor, lut, pe):
    idx_flat = tensor.reshape(N_TOK)
    pe2d = pe[0, :L, :]
    out = _sc_embed(idx_flat, lut, pe2d)
    return out.reshape(B, L, DIM)


# speedup vs baseline: 2.1315x; 1.0044x over previous
import math

import jax
import jax.numpy as jnp
from jax import lax
from jax.experimental import pallas as pl
from jax.experimental.pallas import tpu as pltpu
from jax.experimental.pallas import tpu_sc as plsc

DIM = 128
B = 1024
L = 200
N_TOK = B * L
NC, NS = 2, 16
NW = NC * NS
PER_W = N_TOK // NW
CHUNK = 128
N_CHUNKS = PER_W // CHUNK
NBUF = 5
UNROLL = 4
SCALE = math.sqrt(DIM)
assert N_CHUNKS % NBUF == 0


def _sc_embed(idx_flat, lut, pe2d):
    mesh = plsc.VectorSubcoreMesh(core_axis_name="c", subcore_axis_name="s")

    def body(idx_hbm, lut_hbm, pe_hbm, out_hbm, *scratch):
        idx_v, pe_v = scratch[0], scratch[1]
        gb = scratch[2:2 + NBUF]
        gsem = scratch[2 + NBUF:2 + 2 * NBUF]
        ssem = scratch[2 + 2 * NBUF:2 + 3 * NBUF]
        wid = lax.axis_index("s") * NC + lax.axis_index("c")
        base = wid * PER_W
        pltpu.sync_copy(idx_hbm.at[pl.ds(base, PER_W)], idx_v)
        pltpu.sync_copy(pe_hbm, pe_v)

        def start_gather(j, b):
            pltpu.async_copy(
                lut_hbm.at[idx_v.at[pl.ds(j * CHUNK, CHUNK)]], gb[b], gsem[b])

        def wait_gather(b):
            pltpu.make_async_copy(
                lut_hbm.at[idx_v.at[pl.ds(0, CHUNK)]], gb[b], gsem[b]).wait()

        def start_scatter(j, b):
            pltpu.async_copy(
                gb[b], out_hbm.at[pl.ds(base + j * CHUNK, CHUNK)], ssem[b])

        def wait_scatter(b):
            pltpu.make_async_copy(
                gb[b], out_hbm.at[pl.ds(base, CHUNK)], ssem[b]).wait()

        def compute(j, b):
            ph = lax.rem(j * CHUNK, L)
            buf = gb[b]

            @plsc.parallel_loop(0, CHUNK, unroll=UNROLL)
            def _(r):
                lrow = ph + r
                lrow = jnp.where(lrow >= L, lrow - L, lrow)
                for v in range(DIM // 16):
                    sl = pl.ds(v * 16, 16)
                    buf[r, sl] = buf[r, sl] * SCALE + pe_v[lrow, sl]

        for j in range(NBUF - 1):
            start_gather(j, j)

        def ring_body(jr, carry):
            for b in range(NBUF):
                j = jr * NBUF + b
                nb = (b + NBUF - 1) % NBUF
                wait_gather(b)
                compute(j, b)

                @pl.when(j >= 1)
                def _():
                    wait_scatter(nb)

                @pl.when(j + NBUF - 1 < N_CHUNKS)
                def _():
                    start_gather(j + NBUF - 1, nb)

                start_scatter(j, b)
            return carry

        lax.fori_loop(0, N_CHUNKS // NBUF, ring_body, 0)
        wait_scatter((N_CHUNKS - 1) % NBUF)

    run = pl.kernel(
        body,
        out_type=jax.ShapeDtypeStruct((N_TOK, DIM), jnp.float32),
        mesh=mesh,
        scratch_types=(
            [pltpu.VMEM((PER_W,), jnp.int32),
             pltpu.VMEM((L, DIM), jnp.float32)]
            + [pltpu.VMEM((CHUNK, DIM), jnp.float32)] * NBUF
            + [pltpu.SemaphoreType.DMA] * (2 * NBUF)
        ),
    )
    return run(idx_flat, lut, pe2d)


@jax.jit
def kernel(tensor, lut, pe):
    idx_flat = tensor.reshape(N_TOK)
    pe2d = pe[0, :L, :]
    out = _sc_embed(idx_flat, lut, pe2d)
    return out.reshape(B, L, DIM)
